# baseline (speedup 1.0000x reference)
import functools

import jax
import jax.numpy as jnp
import numpy as np
from jax import lax
from jax.experimental import pallas as pl
from jax.experimental.pallas import tpu as pltpu
from jax.experimental.pallas import tpu_sc as plsc

N_CLS = 3
N_PROP = 20000
PAD = 20480
NTILE = 16
PER_TILE = PAD // NTILE
CHUNKS = PER_TILE // 16
NDET = 100
ROWS = PAD // 128
LOGMAX = float(np.log(1000.0 / 16.0))
SCORE_THRESH = 0.05
NMS_THRESH = 0.5


def _prep_body(img_ref, cl_ref, br_ref, pr_ref, out_ref):
    img = img_ref[0, 0]
    l0 = cl_ref[0]
    l1 = cl_ref[1]
    l2 = cl_ref[2]
    m = jnp.maximum(jnp.maximum(l0, l1), l2)
    e0 = jnp.exp(l0 - m)
    e1 = jnp.exp(l1 - m)
    e2 = jnp.exp(l2 - m)
    den = e0 + e1 + e2

    px1 = pr_ref[0]
    py1 = pr_ref[1]
    px2 = pr_ref[2]
    py2 = pr_ref[3]
    wdt = px2 - px1
    hgt = py2 - py1
    cx = px1 + 0.5 * wdt
    cy = py1 + 0.5 * hgt

    r0 = lax.broadcasted_iota(jnp.int32, (ROWS, 128), 0)
    r1 = lax.broadcasted_iota(jnp.int32, (ROWS, 128), 1)
    valid = (r0 * 128 + r1) < N_PROP

    for li in range(2):
        sc = (e1 if li == 0 else e2) / den
        sc = jnp.where(sc > SCORE_THRESH, sc, -1.0)
        sc = jnp.where(valid, sc, -3.0)
        dx = br_ref[4 * li + 0] / 10.0
        dy = br_ref[4 * li + 1] / 10.0
        dw = jnp.minimum(br_ref[4 * li + 2] / 5.0, LOGMAX)
        dh = jnp.minimum(br_ref[4 * li + 3] / 5.0, LOGMAX)
        pcx = dx * wdt + cx
        pcy = dy * hgt + cy
        pw = jnp.exp(dw) * wdt
        ph = jnp.exp(dh) * hgt
        x1 = jnp.clip(pcx - 0.5 * pw, 0.0, img)
        y1 = jnp.clip(pcy - 0.5 * ph, 0.0, img)
        x2 = jnp.clip(pcx + 0.5 * pw, 0.0, img)
        y2 = jnp.clip(pcy + 0.5 * ph, 0.0, img)
        out_ref[li, 0] = x1
        out_ref[li, 1] = y1
        out_ref[li, 2] = x2
        out_ref[li, 3] = y2
        out_ref[li, 4] = (x2 - x1) * (y2 - y1)
        out_ref[li, 5] = sc


def _splat_i(x):
    return jnp.full((16,), x, dtype=jnp.int32)


@functools.partial(
    pl.kernel,
    out_type=jax.ShapeDtypeStruct((2, NDET * 16), jnp.float32),
    mesh=plsc.VectorSubcoreMesh(core_axis_name="c", subcore_axis_name="s"),
    compiler_params=pltpu.CompilerParams(needs_layout_passes=False),
    scratch_types=[
        pltpu.VMEM((5 * PER_TILE,), jnp.float32),
        pltpu.VMEM((PER_TILE,), jnp.float32),
        pltpu.VMEM((16,), jnp.float32),
        pltpu.VMEM((NTILE * 16,), jnp.float32),
        pltpu.VMEM((NDET * 16,), jnp.float32),
        pltpu.VMEM_SHARED((2 * NTILE * 16,), jnp.float32),
    ],
)
def _nms_sc(mega_hbm, out_hbm, boxes_v, score_v, pub_v, loc_v, outb_v, shared_v):
    c = lax.axis_index("c")
    s = lax.axis_index("s")
    base = s * PER_TILE
    for j in range(5):
        pltpu.sync_copy(
            mega_hbm.at[c, j, pl.ds(base, PER_TILE)],
            boxes_v.at[pl.ds(j * PER_TILE, PER_TILE)],
        )
    pltpu.sync_copy(mega_hbm.at[c, 5, pl.ds(base, PER_TILE)], score_v)
    lane = lax.iota(jnp.int32, 16)

    vm0 = score_v[pl.ds(0, 16)]
    ci0 = jnp.zeros((16,), jnp.int32)
    for ch in range(1, CHUNKS):
        v = score_v[pl.ds(ch * 16, 16)]
        upd = v > vm0
        vm0 = jnp.where(upd, v, vm0)
        ci0 = jnp.where(upd, ch, ci0)

    def step(i, carry):
        vm, ci = carry
        m_loc = jnp.max(vm)
        cand = jnp.where(vm == m_loc, ci * 16 + lane, jnp.int32(1 << 30))
        lidx = jnp.min(cand)
        recb = plsc.load_gather(boxes_v, [jnp.minimum(lane, 4) * PER_TILE + lidx])
        recs = plsc.load_gather(score_v, [_splat_i(lidx)])
        pub_v[...] = jnp.where(lane >= 5, recs, recb)
        par = (i & 1) * (NTILE * 16)
        pltpu.sync_copy(pub_v, shared_v.at[pl.ds(par + s * 16, 16)])
        plsc.subcore_barrier()
        pltpu.sync_copy(shared_v.at[pl.ds(par, NTILE * 16)], loc_v)
        tsc = plsc.load_gather(loc_v, [lane * 16 + 5])
        m_g = jnp.max(tsc)
        pos = jnp.min(jnp.where(tsc == m_g, lane, jnp.int32(63)))
        pbase = pos * 16
        wx1 = plsc.load_gather(loc_v, [_splat_i(pbase)])
        wy1 = plsc.load_gather(loc_v, [_splat_i(pbase + 1)])
        wx2 = plsc.load_gather(loc_v, [_splat_i(pbase + 2)])
        wy2 = plsc.load_gather(loc_v, [_splat_i(pbase + 3)])
        war = plsc.load_gather(loc_v, [_splat_i(pbase + 4)])

        @pl.when(pos == s)
        def _mark():
            plsc.store_scatter(
                score_v,
                [_splat_i(lidx)],
                jnp.full((16,), -1.0, jnp.float32),
                mask=lane == 0,
            )

        @pl.when(s == 0)
        def _emit():
            sel = jnp.where(lane >= 4, 5, lane)
            ov = plsc.load_gather(loc_v, [pbase + sel])
            ov = jnp.where(lane < 5, ov, 0.0)
            ov = ov * jnp.where(m_g > 0.0, 1.0, 0.0)
            plsc.store_scatter(outb_v, [i * 16 + lane], ov)

        nvm = jnp.full((16,), -4.0, jnp.float32)
        nci = jnp.zeros((16,), jnp.int32)
        for ch in range(CHUNKS):
            o = ch * 16
            x1 = boxes_v[pl.ds(o, 16)]
            y1 = boxes_v[pl.ds(PER_TILE + o, 16)]
            x2 = boxes_v[pl.ds(2 * PER_TILE + o, 16)]
            y2 = boxes_v[pl.ds(3 * PER_TILE + o, 16)]
            ar = (x2 - x1) * (y2 - y1)
            sc = score_v[pl.ds(o, 16)]
            w = jnp.maximum(jnp.minimum(x2, wx2) - jnp.maximum(x1, wx1), 0.0)
            h = jnp.maximum(jnp.minimum(y2, wy2) - jnp.maximum(y1, wy1), 0.0)
            inter = w * h
            iou = inter / (war + ar - inter + 1e-6)
            nsc = jnp.where(iou >= NMS_THRESH, -1.0, sc)
            score_v[pl.ds(o, 16)] = nsc
            upd = nsc > nvm
            nvm = jnp.where(upd, nsc, nvm)
            nci = jnp.where(upd, ch, nci)
        return nvm, nci

    lax.fori_loop(0, NDET, step, (vm0, ci0))

    @pl.when(s == 0)
    def _flush():
        pltpu.sync_copy(outb_v, out_hbm.at[c])


def kernel(class_logit, box_regression, proposals, image_shape):
    padn = PAD - N_PROP
    cl = jnp.pad(class_logit.astype(jnp.float32), ((0, padn), (0, 0)))
    br = jnp.pad(box_regression.astype(jnp.float32), ((0, padn), (0, 0)))
    pr = jnp.pad(proposals.astype(jnp.float32), ((0, padn), (0, 0)))
    clt = cl.T.reshape(N_CLS, ROWS, 128)
    brt = br.T.reshape(N_CLS * 4, ROWS, 128)[4:]
    prt = pr.T.reshape(4, ROWS, 128)
    img = jnp.asarray(image_shape, jnp.float32).reshape(1, 1)
    mega = pl.pallas_call(
        _prep_body,
        out_shape=jax.ShapeDtypeStruct((2, 6, ROWS, 128), jnp.float32),
    )(img, clt, brt, prt)
    out = _nms_sc(mega.reshape(2, 6, PAD))
    return out.reshape(2, NDET, 16)[:, :, :5].reshape(2 * NDET, 5)


# speedup vs baseline: 1.0563x; 1.0563x over previous
import functools

import jax
import jax.numpy as jnp
import numpy as np
from jax import lax
from jax.experimental import pallas as pl
from jax.experimental.pallas import tpu as pltpu
from jax.experimental.pallas import tpu_sc as plsc

N_CLS = 3
N_PROP = 20000
PAD = 20480
NTILE = 16
PER_TILE = PAD // NTILE
CHUNKS = PER_TILE // 16
NDET = 100
ROWS = PAD // 128
LOGMAX = float(np.log(1000.0 / 16.0))
SCORE_THRESH = 0.05
NMS_THRESH = 0.5


def _prep_body(img_ref, cl_ref, br_ref, pr_ref, out_ref):
    img = img_ref[0, 0]
    l0 = cl_ref[0]
    l1 = cl_ref[1]
    l2 = cl_ref[2]
    m = jnp.maximum(jnp.maximum(l0, l1), l2)
    e0 = jnp.exp(l0 - m)
    e1 = jnp.exp(l1 - m)
    e2 = jnp.exp(l2 - m)
    den = e0 + e1 + e2

    px1 = pr_ref[0]
    py1 = pr_ref[1]
    px2 = pr_ref[2]
    py2 = pr_ref[3]
    wdt = px2 - px1
    hgt = py2 - py1
    cx = px1 + 0.5 * wdt
    cy = py1 + 0.5 * hgt

    r0 = lax.broadcasted_iota(jnp.int32, (ROWS, 128), 0)
    r1 = lax.broadcasted_iota(jnp.int32, (ROWS, 128), 1)
    valid = (r0 * 128 + r1) < N_PROP

    for li in range(2):
        sc = (e1 if li == 0 else e2) / den
        sc = jnp.where(sc > SCORE_THRESH, sc, -1.0)
        sc = jnp.where(valid, sc, -3.0)
        dx = br_ref[4 * li + 0] / 10.0
        dy = br_ref[4 * li + 1] / 10.0
        dw = jnp.minimum(br_ref[4 * li + 2] / 5.0, LOGMAX)
        dh = jnp.minimum(br_ref[4 * li + 3] / 5.0, LOGMAX)
        pcx = dx * wdt + cx
        pcy = dy * hgt + cy
        pw = jnp.exp(dw) * wdt
        ph = jnp.exp(dh) * hgt
        x1 = jnp.clip(pcx - 0.5 * pw, 0.0, img)
        y1 = jnp.clip(pcy - 0.5 * ph, 0.0, img)
        x2 = jnp.clip(pcx + 0.5 * pw, 0.0, img)
        y2 = jnp.clip(pcy + 0.5 * ph, 0.0, img)
        out_ref[li, 0] = x1
        out_ref[li, 1] = y1
        out_ref[li, 2] = x2
        out_ref[li, 3] = y2
        out_ref[li, 4] = (x2 - x1) * (y2 - y1)
        out_ref[li, 5] = sc


def _splat_i(x):
    return jnp.full((16,), x, dtype=jnp.int32)


@functools.partial(
    pl.kernel,
    out_type=jax.ShapeDtypeStruct((2, NDET * 16), jnp.float32),
    mesh=plsc.VectorSubcoreMesh(core_axis_name="c", subcore_axis_name="s"),
    compiler_params=pltpu.CompilerParams(needs_layout_passes=False),
    scratch_types=[
        pltpu.VMEM((5 * PER_TILE,), jnp.float32),
        pltpu.VMEM((PER_TILE,), jnp.float32),
        pltpu.VMEM((16,), jnp.float32),
        pltpu.VMEM((NTILE * 16,), jnp.float32),
        pltpu.VMEM((NDET * 16,), jnp.float32),
        pltpu.VMEM_SHARED((2 * NTILE * 16,), jnp.float32),
    ],
)
def _nms_sc(mega_hbm, out_hbm, boxes_v, score_v, pub_v, loc_v, outb_v, shared_v):
    c = lax.axis_index("c")
    s = lax.axis_index("s")
    base = s * PER_TILE
    for j in range(5):
        pltpu.sync_copy(
            mega_hbm.at[c, j, pl.ds(base, PER_TILE)],
            boxes_v.at[pl.ds(j * PER_TILE, PER_TILE)],
        )
    pltpu.sync_copy(mega_hbm.at[c, 5, pl.ds(base, PER_TILE)], score_v)
    lane = lax.iota(jnp.int32, 16)

    vm0 = score_v[pl.ds(0, 16)]
    ci0 = jnp.zeros((16,), jnp.int32)
    for ch in range(1, CHUNKS):
        v = score_v[pl.ds(ch * 16, 16)]
        upd = v > vm0
        vm0 = jnp.where(upd, v, vm0)
        ci0 = jnp.where(upd, ch, ci0)

    def step(i, carry):
        vm, ci = carry
        m_loc = jnp.max(vm)
        cand = jnp.where(vm == m_loc, ci * 16 + lane, jnp.int32(1 << 30))
        lidx = jnp.min(cand)
        recb = plsc.load_gather(boxes_v, [jnp.minimum(lane, 4) * PER_TILE + lidx])
        recs = plsc.load_gather(score_v, [_splat_i(lidx)])
        pub_v[...] = jnp.where(lane >= 5, recs, recb)
        par = (i & 1) * (NTILE * 16)
        pltpu.sync_copy(pub_v, shared_v.at[pl.ds(par + s * 16, 16)])
        plsc.subcore_barrier()
        pltpu.sync_copy(shared_v.at[pl.ds(par, NTILE * 16)], loc_v)
        tsc = plsc.load_gather(loc_v, [lane * 16 + 5])
        m_g = jnp.max(tsc)
        pos = jnp.min(jnp.where(tsc == m_g, lane, jnp.int32(63)))
        pbase = pos * 16
        wx1 = plsc.load_gather(loc_v, [_splat_i(pbase)])
        wy1 = plsc.load_gather(loc_v, [_splat_i(pbase + 1)])
        wx2 = plsc.load_gather(loc_v, [_splat_i(pbase + 2)])
        wy2 = plsc.load_gather(loc_v, [_splat_i(pbase + 3)])
        war = plsc.load_gather(loc_v, [_splat_i(pbase + 4)])

        @pl.when(pos == s)
        def _mark():
            plsc.store_scatter(
                score_v,
                [_splat_i(lidx)],
                jnp.full((16,), -1.0, jnp.float32),
                mask=lane == 0,
            )

        @pl.when(s == 0)
        def _emit():
            sel = jnp.where(lane >= 4, 5, lane)
            ov = plsc.load_gather(loc_v, [pbase + sel])
            ov = jnp.where(lane < 5, ov, 0.0)
            ov = ov * jnp.where(m_g > 0.0, 1.0, 0.0)
            plsc.store_scatter(outb_v, [i * 16 + lane], ov)

        nvm = jnp.full((16,), -4.0, jnp.float32)
        nci = jnp.zeros((16,), jnp.int32)
        for ch in range(CHUNKS):
            o = ch * 16
            x1 = boxes_v[pl.ds(o, 16)]
            y1 = boxes_v[pl.ds(PER_TILE + o, 16)]
            x2 = boxes_v[pl.ds(2 * PER_TILE + o, 16)]
            y2 = boxes_v[pl.ds(3 * PER_TILE + o, 16)]
            ar = boxes_v[pl.ds(4 * PER_TILE + o, 16)]
            sc = score_v[pl.ds(o, 16)]
            w = jnp.maximum(jnp.minimum(x2, wx2) - jnp.maximum(x1, wx1), 0.0)
            h = jnp.maximum(jnp.minimum(y2, wy2) - jnp.maximum(y1, wy1), 0.0)
            inter = w * h
            iou = inter / (war + ar - inter + 1e-6)
            nsc = jnp.where(iou >= NMS_THRESH, -1.0, sc)
            score_v[pl.ds(o, 16)] = nsc
            upd = nsc > nvm
            nvm = jnp.where(upd, nsc, nvm)
            nci = jnp.where(upd, ch, nci)
        return nvm, nci

    lax.fori_loop(0, NDET, step, (vm0, ci0))

    @pl.when(s == 0)
    def _flush():
        pltpu.sync_copy(outb_v, out_hbm.at[c])


def kernel(class_logit, box_regression, proposals, image_shape):
    padn = PAD - N_PROP
    cl = jnp.pad(class_logit.astype(jnp.float32), ((0, padn), (0, 0)))
    br = jnp.pad(box_regression.astype(jnp.float32), ((0, padn), (0, 0)))
    pr = jnp.pad(proposals.astype(jnp.float32), ((0, padn), (0, 0)))
    clt = cl.T.reshape(N_CLS, ROWS, 128)
    brt = br.T.reshape(N_CLS * 4, ROWS, 128)[4:]
    prt = pr.T.reshape(4, ROWS, 128)
    img = jnp.asarray(image_shape, jnp.float32).reshape(1, 1)
    mega = pl.pallas_call(
        _prep_body,
        out_shape=jax.ShapeDtypeStruct((2, 6, ROWS, 128), jnp.float32),
    )(img, clt, brt, prt)
    out = _nms_sc(mega.reshape(2, 6, PAD))
    return out.reshape(2, NDET, 16)[:, :, :5].reshape(2 * NDET, 5)
